# baseline (speedup 1.0000x reference)
import functools

import jax
import jax.numpy as jnp
from jax import lax
from jax.experimental import pallas as pl
from jax.experimental.pallas import tpu as pltpu
from jax.experimental.pallas import tpu_sc as plsc

N = 25000
NP = 25600
E = 400000
D_IN = 128
C = 32
OUT = 16
W = 40
WT = 32
CHUNK = 128
NS = 16
EPT = E // NS
CNT = 196
EGRP = 256
RPT = NP // NS
GP = 16
NPA = 25088
RPTA = NPA // NS
NT = 25088
F32 = jnp.float32
I32 = jnp.int32


def _proj_body(xsrc_ref, xdst_ref, Ws_ref, asv_ref, Wd_ref, ad_ref,
               hs_ref, asrc_ref, adst_ref, amax_ref, mscr):
    i = pl.program_id(1)
    x = xsrc_ref[0]
    hs = jnp.dot(x, Ws_ref[0], preferred_element_type=F32)
    asrc = jnp.sum(hs * asv_ref[0, 0][None, :], axis=1)
    rows = (i * hs.shape[0]
            + jax.lax.broadcasted_iota(I32, (hs.shape[0],), 0))
    asrc = jnp.where(rows < N, asrc, -3e38)
    hs_ref[0] = hs
    asrc_ref[0, 0] = asrc
    wdv = jnp.sum(Wd_ref[0] * ad_ref[0, 0][None, :], axis=1)
    adst_ref[0, 0] = jnp.sum(xdst_ref[0] * wdv[None, :], axis=1)
    bm = jnp.max(asrc)

    @pl.when(i == 0)
    def _():
        mscr[0] = bm

    @pl.when(i > 0)
    def _():
        mscr[0] = jnp.maximum(mscr[0], bm)

    amax_ref[0, 0] = jnp.full((128,), mscr[0], F32)


def _make_proj(d_in, nb, swap):
    b = NP // nb
    s = 1 if swap else 0
    return pl.pallas_call(
        _proj_body,
        grid=(2, nb),
        in_specs=[
            pl.BlockSpec((1, b, d_in), lambda r, i: (s - r if s else r, i, 0)),
            pl.BlockSpec((1, b, d_in), lambda r, i: (r if s else 1 - r, i, 0)),
            pl.BlockSpec((1, d_in, C), lambda r, i: (r, 0, 0)),
            pl.BlockSpec((1, 1, C), lambda r, i: (r, 0, 0)),
            pl.BlockSpec((1, d_in, C), lambda r, i: (r, 0, 0)),
            pl.BlockSpec((1, 1, C), lambda r, i: (r, 0, 0)),
        ],
        out_specs=[
            pl.BlockSpec((1, b, WT), lambda r, i: (r, i, 0)),
            pl.BlockSpec((1, 1, b), lambda r, i: (r, 0, i)),
            pl.BlockSpec((1, 1, b), lambda r, i: (r, 0, i)),
            pl.BlockSpec((1, 1, 128), lambda r, i: (r, 0, 0)),
        ],
        out_shape=[
            jax.ShapeDtypeStruct((2, NP, WT), F32),
            jax.ShapeDtypeStruct((2, 1, NP), F32),
            jax.ShapeDtypeStruct((2, 1, NP), F32),
            jax.ShapeDtypeStruct((2, 1, 128), F32),
        ],
        scratch_shapes=[pltpu.SMEM((1,), F32)],
    )


def _final_body(x_ref, w0_ref, b0_ref, w1_ref, b1_ref, out_ref):
    x = x_ref[0]
    y = jnp.dot(x, w0_ref[0], preferred_element_type=F32) + b0_ref[0, 0][None, :]
    out_ref[0] = (jnp.dot(y, w1_ref[0], preferred_element_type=F32)
                  + b1_ref[0, 0][None, :])


def _make_final(nb):
    b = NP // nb
    return pl.pallas_call(
        _final_body,
        grid=(2, nb),
        in_specs=[
            pl.BlockSpec((1, b, C), lambda t, i: (1 - t, i, 0)),
            pl.BlockSpec((1, C, C), lambda t, i: (t, 0, 0)),
            pl.BlockSpec((1, 1, C), lambda t, i: (t, 0, 0)),
            pl.BlockSpec((1, C, OUT), lambda t, i: (t, 0, 0)),
            pl.BlockSpec((1, 1, OUT), lambda t, i: (t, 0, 0)),
        ],
        out_specs=[pl.BlockSpec((1, b, OUT), lambda t, i: (t, i, 0))],
        out_shape=[jax.ShapeDtypeStruct((2, NP, OUT), F32)],
    )


_MESH = plsc.VectorSubcoreMesh(core_axis_name="c", subcore_axis_name="s",
                               num_cores=2, num_subcores=NS)


@functools.partial(
    pl.kernel,
    out_type=jax.ShapeDtypeStruct((2, NP, C), F32),
    mesh=_MESH,
    compiler_params=pltpu.CompilerParams(use_tc_tiling_on_sc=False,
                                         needs_layout_passes=False),
    scratch_types=[
        pltpu.VMEM((NT,), F32),
        pltpu.VMEM((NT,), F32),
        pltpu.VMEM((C,), F32),
        pltpu.VMEM((16,), F32),
        pltpu.VMEM((EGRP, 2), I32),
        pltpu.VMEM((2, CHUNK), I32),
        pltpu.VMEM((2, CHUNK), I32),
        pltpu.VMEM((2, CHUNK, WT), F32),
        pltpu.VMEM((CHUNK, W), F32),
        pltpu.VMEM_SHARED((NPA, W), F32),
        pltpu.SemaphoreType.DMA((2,)),
    ],
)
def _edge_kernel(edges_hbm, asrc_hbm, adst_hbm, bias_hbm, amax_hbm, hs_hbm,
                 zeros_hbm, xout_hbm,
                 asrc_loc, adst_loc, bias_loc, amax_loc, ebuf, srcoff, dstc,
                 rows_g, rows_s, accum, gsem):
    r = lax.axis_index("c")
    sid = lax.axis_index("s")
    row0 = sid * RPTA

    pltpu.sync_copy(zeros_hbm.at[pl.ds(row0, RPTA)],
                    accum.at[pl.ds(row0, RPTA)])
    pltpu.sync_copy(asrc_hbm.at[r, pl.ds(0, NT)], asrc_loc)
    pltpu.sync_copy(adst_hbm.at[r, pl.ds(0, NT)], adst_loc)
    pltpu.sync_copy(bias_hbm.at[r], bias_loc)
    pltpu.sync_copy(amax_hbm.at[r, pl.ds(0, 16)], amax_loc)

    iota16 = lax.iota(I32, 16)
    z16 = jnp.zeros((16,), I32)
    one16 = jnp.full((16,), 1, I32)
    col_den = jnp.full((16,), C, I32)
    a_maxv = plsc.load_gather(amax_loc, [z16])
    roffv = jnp.full((16,), r * NP, I32)
    estart = sid * EPT

    plsc.subcore_barrier()

    def _build(kk, gstart, slot, sl16):
        off = estart + kk * CHUNK - gstart
        for g in range(CHUNK // 16):
            ri = jnp.minimum(jnp.full((16,), g * 16, I32) + iota16 + off,
                             EGRP - 1)
            sv = plsc.load_gather(ebuf, [ri, z16])
            dv = plsc.load_gather(ebuf, [ri, one16])
            gi = jnp.full((16,), g * 16, I32) + iota16
            plsc.store_scatter(srcoff, [sl16, gi], sv + roffv)
            plsc.store_scatter(dstc, [sl16, gi], dv)
        pltpu.async_copy(hs_hbm.at[srcoff.at[slot]], rows_g.at[slot],
                         gsem.at[slot])

    gs0 = jnp.minimum(jnp.asarray(estart, I32), E - EGRP)
    pltpu.sync_copy(edges_hbm.at[r, pl.ds(gs0, EGRP)], ebuf)
    _build(0, gs0, 0, z16)

    def _chunk(k, gstart):
        slot = lax.rem(k, 2)
        nslot = 1 - slot

        ng = jnp.minimum(estart + (k + 1) * CHUNK, E - EGRP)
        refill = lax.rem(k + 1, EGRP // CHUNK) == 0
        new_gstart = jnp.where(refill, ng, gstart)

        @pl.when(jnp.logical_and(k + 1 < CNT, refill))
        def _():
            pltpu.sync_copy(edges_hbm.at[r, pl.ds(ng, EGRP)], ebuf)

        @pl.when(k + 1 < CNT)
        def _():
            _build(k + 1, new_gstart, nslot, jnp.full((16,), nslot, I32))

        pltpu.make_async_copy(hs_hbm.at[srcoff.at[slot]], rows_g.at[slot],
                              gsem.at[slot]).wait()
        sl16 = jnp.full((16,), slot, I32)
        for g in range(CHUNK // 16):
            gi = jnp.full((16,), g * 16, I32) + iota16
            dv = plsc.load_gather(dstc, [sl16, gi])
            a_d = plsc.load_gather(adst_loc, [dv])
            sv = plsc.load_gather(srcoff, [sl16, gi]) - roffv
            a_s = plsc.load_gather(asrc_loc, [sv])
            s = a_s + a_d
            act = jnp.maximum(s, 0.2 * s)
            t = a_maxv + a_d
            mp = jnp.maximum(t, 0.2 * t)
            ex = jnp.exp(act - mp)
            lid = jnp.full((16,), k * CHUNK + g * 16, I32) + iota16
            ex = jnp.where(lid < EPT, ex, 0.0)
            for c in range(C):
                cv = jnp.full((16,), c, I32)
                hv = plsc.load_gather(rows_g, [sl16, gi, cv])
                plsc.store_scatter(rows_s, [gi, cv], hv * ex)
            plsc.store_scatter(rows_s, [gi, col_den], ex)
        pltpu.sync_copy(rows_s, accum.at[dstc.at[slot]], add=True)
        return new_gstart

    lax.fori_loop(0, CNT, _chunk, gs0)
    plsc.subcore_barrier()

    def _post(bk, carry):
        rbase = row0 + bk * GP
        pltpu.sync_copy(accum.at[pl.ds(rbase, GP)], rows_s.at[pl.ds(0, GP)])
        for g in range(GP // 16):
            rid = jnp.full((16,), g * 16, I32) + iota16
            den = plsc.load_gather(rows_s, [rid, col_den]) + 1e-16
            rec = 1.0 / den
            for c in range(C):
                cv = jnp.full((16,), c, I32)
                bc = plsc.load_gather(bias_loc, [cv])
                v = plsc.load_gather(rows_s, [rid, cv]) * rec + bc
                plsc.store_scatter(rows_g, [z16, rid, cv], jnp.maximum(v, 0.0))
        pltpu.sync_copy(rows_g.at[0, pl.ds(0, GP)],
                        xout_hbm.at[r, pl.ds(rbase, GP)])
        return carry

    lax.fori_loop(0, RPTA // GP, _post, 0)


def _stack2(pa, pb, k):
    return jnp.stack([pa[k], pb[k]])


def _stack2v(pa, pb, k):
    return jnp.stack([pa[k], pb[k]])[:, None, :]


def kernel(x_user, x_item, edge_index_u2i, edge_index_i2u, params):
    p = params
    pad = ((0, NP - N), (0, 0))
    xs = jnp.stack([jnp.pad(x_user, pad), jnp.pad(x_item, pad)])

    edges = jnp.stack([edge_index_u2i.T, edge_index_i2u.T])
    zeros = jnp.zeros((NPA, W), F32)

    c0u, c0i = p['c0_u2i'], p['c0_i2u']
    c1u, c1i = p['c1_u2i'], p['c1_i2u']

    hs0, asrc0, adst0, amax0 = _make_proj(D_IN, 25, False)(
        xs, xs, _stack2(c0u, c0i, 'Ws'), _stack2v(c0u, c0i, 'as'),
        _stack2(c0u, c0i, 'Wd'), _stack2v(c0u, c0i, 'ad'))
    x1 = _edge_kernel(edges, asrc0.reshape(2, NP), adst0.reshape(2, NP),
                      _stack2(c0u, c0i, 'b'), amax0.reshape(2, 128),
                      hs0.reshape(2 * NP, WT), zeros)

    hs1, asrc1, adst1, amax1 = _make_proj(C, 25, True)(
        x1, x1, _stack2(c1u, c1i, 'Ws'), _stack2v(c1u, c1i, 'as'),
        _stack2(c1u, c1i, 'Wd'), _stack2v(c1u, c1i, 'ad'))
    x2 = _edge_kernel(edges, asrc1.reshape(2, NP), adst1.reshape(2, NP),
                      _stack2(c1u, c1i, 'b'), amax1.reshape(2, 128),
                      hs1.reshape(2 * NP, WT), zeros)

    outs, = _make_final(25)(
        x2,
        jnp.stack([p['lin0_u_W'], p['lin0_i_W']]),
        jnp.stack([p['lin0_u_b'], p['lin0_i_b']])[:, None, :],
        jnp.stack([p['lin1_u_W'], p['lin1_i_W']]),
        jnp.stack([p['lin1_u_b'], p['lin1_i_b']])[:, None, :])
    return (outs[0, :N], outs[1, :N])


# speedup vs baseline: 1.0529x; 1.0529x over previous
import functools

import jax
import jax.numpy as jnp
from jax import lax
from jax.experimental import pallas as pl
from jax.experimental.pallas import tpu as pltpu
from jax.experimental.pallas import tpu_sc as plsc

N = 25000
NP = 25600
E = 400000
D_IN = 128
C = 32
OUT = 16
W = 40
WT = 48
CHUNK = 256
NS = 16
EPT = E // NS
CNT = 98
EGRP = 512
RPT = NP // NS
GP = 224
NPA = 25088
RPTA = NPA // NS
NT = 25088
F32 = jnp.float32
I32 = jnp.int32


def _proj_body(xsrc_ref, xdst_ref, Ws_ref, asv_ref, Wd_ref, ad_ref,
               hs_ref, asrc_ref, adst_ref, amax_ref, mscr):
    i = pl.program_id(1)
    x = xsrc_ref[0]
    hs = jnp.dot(x, Ws_ref[0], preferred_element_type=F32)
    asrc = jnp.sum(hs * asv_ref[0, 0][None, :], axis=1)
    rows = (i * hs.shape[0]
            + jax.lax.broadcasted_iota(I32, (hs.shape[0],), 0))
    asrc = jnp.where(rows < N, asrc, -3e38)
    hs_ref[0] = jnp.concatenate(
        [hs, asrc[:, None], jnp.zeros((hs.shape[0], WT - C - 1), F32)], axis=1)
    asrc_ref[0, 0] = asrc
    wdv = jnp.sum(Wd_ref[0] * ad_ref[0, 0][None, :], axis=1)
    adst_ref[0, 0] = jnp.sum(xdst_ref[0] * wdv[None, :], axis=1)
    bm = jnp.max(asrc)

    @pl.when(i == 0)
    def _():
        mscr[0] = bm

    @pl.when(i > 0)
    def _():
        mscr[0] = jnp.maximum(mscr[0], bm)

    amax_ref[0, 0] = jnp.full((128,), mscr[0], F32)


def _make_proj(d_in, nb, swap):
    b = NP // nb
    s = 1 if swap else 0
    return pl.pallas_call(
        _proj_body,
        grid=(2, nb),
        in_specs=[
            pl.BlockSpec((1, b, d_in), lambda r, i: (s - r if s else r, i, 0)),
            pl.BlockSpec((1, b, d_in), lambda r, i: (r if s else 1 - r, i, 0)),
            pl.BlockSpec((1, d_in, C), lambda r, i: (r, 0, 0)),
            pl.BlockSpec((1, 1, C), lambda r, i: (r, 0, 0)),
            pl.BlockSpec((1, d_in, C), lambda r, i: (r, 0, 0)),
            pl.BlockSpec((1, 1, C), lambda r, i: (r, 0, 0)),
        ],
        out_specs=[
            pl.BlockSpec((1, b, WT), lambda r, i: (r, i, 0)),
            pl.BlockSpec((1, 1, b), lambda r, i: (r, 0, i)),
            pl.BlockSpec((1, 1, b), lambda r, i: (r, 0, i)),
            pl.BlockSpec((1, 1, 128), lambda r, i: (r, 0, 0)),
        ],
        out_shape=[
            jax.ShapeDtypeStruct((2, NP, WT), F32),
            jax.ShapeDtypeStruct((2, 1, NP), F32),
            jax.ShapeDtypeStruct((2, 1, NP), F32),
            jax.ShapeDtypeStruct((2, 1, 128), F32),
        ],
        scratch_shapes=[pltpu.SMEM((1,), F32)],
    )


def _final_body(x_ref, w0_ref, b0_ref, w1_ref, b1_ref, out_ref):
    x = x_ref[0]
    y = jnp.dot(x, w0_ref[0], preferred_element_type=F32) + b0_ref[0, 0][None, :]
    out_ref[0] = (jnp.dot(y, w1_ref[0], preferred_element_type=F32)
                  + b1_ref[0, 0][None, :])


def _make_final(nb):
    b = NP // nb
    return pl.pallas_call(
        _final_body,
        grid=(2, nb),
        in_specs=[
            pl.BlockSpec((1, b, C), lambda t, i: (1 - t, i, 0)),
            pl.BlockSpec((1, C, C), lambda t, i: (t, 0, 0)),
            pl.BlockSpec((1, 1, C), lambda t, i: (t, 0, 0)),
            pl.BlockSpec((1, C, OUT), lambda t, i: (t, 0, 0)),
            pl.BlockSpec((1, 1, OUT), lambda t, i: (t, 0, 0)),
        ],
        out_specs=[pl.BlockSpec((1, b, OUT), lambda t, i: (t, i, 0))],
        out_shape=[jax.ShapeDtypeStruct((2, NP, OUT), F32)],
    )


_MESH = plsc.VectorSubcoreMesh(core_axis_name="c", subcore_axis_name="s",
                               num_cores=2, num_subcores=NS)


@functools.partial(
    pl.kernel,
    out_type=jax.ShapeDtypeStruct((2, NP, C), F32),
    mesh=_MESH,
    compiler_params=pltpu.CompilerParams(use_tc_tiling_on_sc=False,
                                         needs_layout_passes=False),
    scratch_types=[
        pltpu.VMEM((NT,), F32),
        pltpu.VMEM((C,), F32),
        pltpu.VMEM((16,), F32),
        pltpu.VMEM((EGRP, 2), I32),
        pltpu.VMEM((1, CHUNK), I32),
        pltpu.VMEM((1, CHUNK), I32),
        pltpu.VMEM((CHUNK, WT), F32),
        pltpu.VMEM((CHUNK, W), F32),
        pltpu.VMEM((GP, C), F32),
        pltpu.VMEM_SHARED((NPA, W), F32),
        pltpu.SemaphoreType.DMA,
    ],
)
def _edge_kernel(edges_hbm, adst_hbm, bias_hbm, amax_hbm, hs_hbm,
                 zeros_hbm, xout_hbm,
                 adst_loc, bias_loc, amax_loc, ebuf, srcoff, dstc,
                 rows_g, rows_s, pstage, accum, gsem):
    r = lax.axis_index("c")
    sid = lax.axis_index("s")
    row0 = sid * RPTA

    pltpu.sync_copy(zeros_hbm.at[pl.ds(row0, RPTA)],
                    accum.at[pl.ds(row0, RPTA)])
    pltpu.sync_copy(adst_hbm.at[r, pl.ds(0, NT)], adst_loc)
    pltpu.sync_copy(bias_hbm.at[r], bias_loc)
    pltpu.sync_copy(amax_hbm.at[r, pl.ds(0, 16)], amax_loc)

    iota16 = lax.iota(I32, 16)
    z16 = jnp.zeros((16,), I32)
    one16 = jnp.full((16,), 1, I32)
    col_den = jnp.full((16,), C, I32)
    a_maxv = plsc.load_gather(amax_loc, [z16])
    roffv = jnp.full((16,), r * NP, I32)
    estart = sid * EPT

    plsc.subcore_barrier()

    def _chunk(k, gstart):
        ng = jnp.minimum(estart + k * CHUNK, E - EGRP)
        refill = lax.rem(k, EGRP // CHUNK) == 0
        new_gstart = jnp.where(refill, ng, gstart)

        @pl.when(refill)
        def _():
            pltpu.sync_copy(edges_hbm.at[r, pl.ds(ng, EGRP)], ebuf)

        off = estart + k * CHUNK - new_gstart
        for g in range(CHUNK // 16):
            ri = jnp.minimum(jnp.full((16,), g * 16, I32) + iota16 + off,
                             EGRP - 1)
            sv = plsc.load_gather(ebuf, [ri, z16])
            dv = plsc.load_gather(ebuf, [ri, one16])
            gi = jnp.full((16,), g * 16, I32) + iota16
            plsc.store_scatter(srcoff, [z16, gi], sv + roffv)
            plsc.store_scatter(dstc, [z16, gi], dv)
        pltpu.async_copy(hs_hbm.at[srcoff.at[0]], rows_g, gsem).wait()

        for g in range(CHUNK // 16):
            gi = jnp.full((16,), g * 16, I32) + iota16
            dv = plsc.load_gather(dstc, [z16, gi])
            a_d = plsc.load_gather(adst_loc, [dv])
            a_s = plsc.load_gather(rows_g, [gi, col_den])
            sc = a_s + a_d
            act = jnp.maximum(sc, 0.2 * sc)
            t = a_maxv + a_d
            mp = jnp.maximum(t, 0.2 * t)
            ex = jnp.exp(act - mp)
            lid = jnp.full((16,), k * CHUNK + g * 16, I32) + iota16
            ex = jnp.where(lid < EPT, ex, 0.0)
            for c in range(C):
                cv = jnp.full((16,), c, I32)
                hv = plsc.load_gather(rows_g, [gi, cv])
                plsc.store_scatter(rows_s, [gi, cv], hv * ex)
            plsc.store_scatter(rows_s, [gi, col_den], ex)
        pltpu.sync_copy(rows_s, accum.at[dstc.at[0]], add=True)
        return new_gstart

    lax.fori_loop(0, CNT, _chunk, jnp.asarray(0, I32))
    plsc.subcore_barrier()

    def _post(bk, carry):
        rbase = row0 + bk * GP
        pltpu.sync_copy(accum.at[pl.ds(rbase, GP)], rows_s.at[pl.ds(0, GP)])
        for g in range(GP // 16):
            rid = jnp.full((16,), g * 16, I32) + iota16
            den = plsc.load_gather(rows_s, [rid, col_den]) + 1e-16
            rec = 1.0 / den
            for c in range(C):
                cv = jnp.full((16,), c, I32)
                bc = plsc.load_gather(bias_loc, [cv])
                v = plsc.load_gather(rows_s, [rid, cv]) * rec + bc
                plsc.store_scatter(pstage, [rid, cv], jnp.maximum(v, 0.0))
        pltpu.sync_copy(pstage, xout_hbm.at[r, pl.ds(rbase, GP)])
        return carry

    lax.fori_loop(0, RPTA // GP, _post, 0)


def _stack2(pa, pb, k):
    return jnp.stack([pa[k], pb[k]])


def _stack2v(pa, pb, k):
    return jnp.stack([pa[k], pb[k]])[:, None, :]


def kernel(x_user, x_item, edge_index_u2i, edge_index_i2u, params):
    p = params
    pad = ((0, NP - N), (0, 0))
    xs = jnp.stack([jnp.pad(x_user, pad), jnp.pad(x_item, pad)])

    edges = jnp.stack([edge_index_u2i.T, edge_index_i2u.T])
    zeros = jnp.zeros((NPA, W), F32)

    c0u, c0i = p['c0_u2i'], p['c0_i2u']
    c1u, c1i = p['c1_u2i'], p['c1_i2u']

    hs0, asrc0, adst0, amax0 = _make_proj(D_IN, 25, False)(
        xs, xs, _stack2(c0u, c0i, 'Ws'), _stack2v(c0u, c0i, 'as'),
        _stack2(c0u, c0i, 'Wd'), _stack2v(c0u, c0i, 'ad'))
    x1 = _edge_kernel(edges, adst0.reshape(2, NP), _stack2(c0u, c0i, 'b'),
                      amax0.reshape(2, 128), hs0.reshape(2 * NP, WT), zeros)

    hs1, asrc1, adst1, amax1 = _make_proj(C, 25, True)(
        x1, x1, _stack2(c1u, c1i, 'Ws'), _stack2v(c1u, c1i, 'as'),
        _stack2(c1u, c1i, 'Wd'), _stack2v(c1u, c1i, 'ad'))
    x2 = _edge_kernel(edges, adst1.reshape(2, NP), _stack2(c1u, c1i, 'b'),
                      amax1.reshape(2, 128), hs1.reshape(2 * NP, WT), zeros)

    outs, = _make_final(25)(
        x2,
        jnp.stack([p['lin0_u_W'], p['lin0_i_W']]),
        jnp.stack([p['lin0_u_b'], p['lin0_i_b']])[:, None, :],
        jnp.stack([p['lin1_u_W'], p['lin1_i_W']]),
        jnp.stack([p['lin1_u_b'], p['lin1_i_b']])[:, None, :])
    return (outs[0, :N], outs[1, :N])


# speedup vs baseline: 1.2940x; 1.2290x over previous
import functools

import jax
import jax.numpy as jnp
from jax import lax
from jax.experimental import pallas as pl
from jax.experimental.pallas import tpu as pltpu
from jax.experimental.pallas import tpu_sc as plsc

N = 25000
NP = 25600
E = 400000
D_IN = 128
C = 32
OUT = 16
W = 40
CHUNK = 128
NCHUNK = E // CHUNK
NS = 16
RPT = NP // NS
GP = 80
F32 = jnp.float32
I32 = jnp.int32


def _proj_body(xsrc_ref, xdst_ref, Ws_ref, asv_ref, Wd_ref, ad_ref,
               hs_ref, asrc_ref, adst_ref):
    x = xsrc_ref[0]
    hs = jnp.dot(x, Ws_ref[0], preferred_element_type=F32)
    hs_ref[0] = hs
    asrc_ref[0, 0] = jnp.sum(hs * asv_ref[0, 0][None, :], axis=1)
    wdv = jnp.sum(Wd_ref[0] * ad_ref[0, 0][None, :], axis=1)
    adst_ref[0, 0] = jnp.sum(xdst_ref[0] * wdv[None, :], axis=1)


def _make_proj(d_in, nb, swap):
    b = NP // nb
    s = 1 if swap else 0
    return pl.pallas_call(
        _proj_body,
        grid=(2, nb),
        in_specs=[
            pl.BlockSpec((1, b, d_in), lambda r, i: (s - r if s else r, i, 0)),
            pl.BlockSpec((1, b, d_in), lambda r, i: (r if s else 1 - r, i, 0)),
            pl.BlockSpec((1, d_in, C), lambda r, i: (r, 0, 0)),
            pl.BlockSpec((1, 1, C), lambda r, i: (r, 0, 0)),
            pl.BlockSpec((1, d_in, C), lambda r, i: (r, 0, 0)),
            pl.BlockSpec((1, 1, C), lambda r, i: (r, 0, 0)),
        ],
        out_specs=[
            pl.BlockSpec((1, b, C), lambda r, i: (r, i, 0)),
            pl.BlockSpec((1, 1, b), lambda r, i: (r, 0, i)),
            pl.BlockSpec((1, 1, b), lambda r, i: (r, 0, i)),
        ],
        out_shape=[
            jax.ShapeDtypeStruct((2, NP, C), F32),
            jax.ShapeDtypeStruct((2, 1, NP), F32),
            jax.ShapeDtypeStruct((2, 1, NP), F32),
        ],
    )


def _final_body(x_ref, w0_ref, b0_ref, w1_ref, b1_ref, out_ref):
    x = x_ref[0]
    y = jnp.dot(x, w0_ref[0], preferred_element_type=F32) + b0_ref[0, 0][None, :]
    out_ref[0] = (jnp.dot(y, w1_ref[0], preferred_element_type=F32)
                  + b1_ref[0, 0][None, :])


def _make_final(nb):
    b = NP // nb
    return pl.pallas_call(
        _final_body,
        grid=(2, nb),
        in_specs=[
            pl.BlockSpec((1, b, C), lambda t, i: (1 - t, i, 0)),
            pl.BlockSpec((1, C, C), lambda t, i: (t, 0, 0)),
            pl.BlockSpec((1, 1, C), lambda t, i: (t, 0, 0)),
            pl.BlockSpec((1, C, OUT), lambda t, i: (t, 0, 0)),
            pl.BlockSpec((1, 1, OUT), lambda t, i: (t, 0, 0)),
        ],
        out_specs=[pl.BlockSpec((1, b, OUT), lambda t, i: (t, i, 0))],
        out_shape=[jax.ShapeDtypeStruct((2, NP, OUT), F32)],
    )


_MESH = plsc.VectorSubcoreMesh(core_axis_name="c", subcore_axis_name="s",
                               num_cores=2, num_subcores=NS)


@functools.partial(
    pl.kernel,
    out_type=jax.ShapeDtypeStruct((2, NP, C), F32),
    mesh=_MESH,
    compiler_params=pltpu.CompilerParams(use_tc_tiling_on_sc=False,
                                         needs_layout_passes=False),
    scratch_types=[
        pltpu.VMEM((NP,), F32),
        pltpu.VMEM((NP,), F32),
        pltpu.VMEM((C,), F32),
        pltpu.VMEM((1, CHUNK), I32),
        pltpu.VMEM((1, CHUNK), I32),
        pltpu.VMEM((1, CHUNK), I32),
        pltpu.VMEM((CHUNK, C), F32),
        pltpu.VMEM((CHUNK, W), F32),
        pltpu.VMEM_SHARED((NP, W), F32),
        pltpu.SemaphoreType.DMA,
    ],
)
def _edge_kernel(src_hbm, dst_hbm, asrc_hbm, adst_hbm, bias_hbm, hs_hbm,
                 zeros_hbm, xout_hbm,
                 asrc_loc, adst_loc, bias_loc, srcbuf, dstbuf, srcoff,
                 rows_g, rows_s, accum, sem):
    r = lax.axis_index("c")
    sid = lax.axis_index("s")
    row0 = sid * RPT

    pltpu.sync_copy(zeros_hbm.at[pl.ds(row0, RPT)], accum.at[pl.ds(row0, RPT)])
    pltpu.sync_copy(asrc_hbm.at[r], asrc_loc)
    pltpu.sync_copy(adst_hbm.at[r], adst_loc)
    pltpu.sync_copy(bias_hbm.at[r], bias_loc)

    def _mx(i, m):
        return jnp.maximum(m, asrc_loc[pl.ds(i * 16, 16)])
    mvec = lax.fori_loop(0, NP // 16, _mx, jnp.full((16,), -jnp.inf, F32))
    rows_s[0, pl.ds(0, 16)] = mvec
    z16 = jnp.zeros((16,), I32)
    a_maxv = plsc.load_gather(rows_s, [z16, z16])
    for c in range(1, 16):
        a_maxv = jnp.maximum(
            a_maxv, plsc.load_gather(rows_s, [z16, jnp.full((16,), c, I32)]))

    plsc.subcore_barrier()

    iota16 = lax.iota(I32, 16)
    col_den = jnp.full((16,), C, I32)
    roff = (r * NP).astype(I32)

    nchunks = jnp.where(sid < NCHUNK % NS, NCHUNK // NS + 1, NCHUNK // NS)

    def _chunk(k, carry):
        base = (sid + k * NS) * CHUNK
        pltpu.sync_copy(src_hbm.at[r, pl.ds(base, CHUNK)], srcbuf.at[0])
        pltpu.sync_copy(dst_hbm.at[r, pl.ds(base, CHUNK)], dstbuf.at[0])
        for g in range(CHUNK // 16):
            sv = srcbuf[0, pl.ds(g * 16, 16)]
            srcoff[0, pl.ds(g * 16, 16)] = sv + roff
        pltpu.async_copy(hs_hbm.at[srcoff.at[0]], rows_g, sem).wait()
        for g in range(CHUNK // 16):
            sv = srcbuf[0, pl.ds(g * 16, 16)]
            dv = dstbuf[0, pl.ds(g * 16, 16)]
            a_s = plsc.load_gather(asrc_loc, [sv])
            a_d = plsc.load_gather(adst_loc, [dv])
            s = a_s + a_d
            act = jnp.maximum(s, 0.2 * s)
            t = a_maxv + a_d
            mp = jnp.maximum(t, 0.2 * t)
            ex = jnp.exp(act - mp)
            eidx = jnp.full((16,), g * 16, I32) + iota16
            for c in range(C):
                cv = jnp.full((16,), c, I32)
                hv = plsc.load_gather(rows_g, [eidx, cv])
                plsc.store_scatter(rows_s, [eidx, cv], hv * ex)
            plsc.store_scatter(rows_s, [eidx, col_den], ex)
        pltpu.sync_copy(rows_s, accum.at[dstbuf.at[0]], add=True)
        return carry

    lax.fori_loop(0, nchunks, _chunk, 0)
    plsc.subcore_barrier()

    def _post(bk, carry):
        rbase = row0 + bk * GP
        pltpu.sync_copy(accum.at[pl.ds(rbase, GP)], rows_s.at[pl.ds(0, GP)])
        for g in range(GP // 16):
            rid = jnp.full((16,), g * 16, I32) + iota16
            den = plsc.load_gather(rows_s, [rid, col_den]) + 1e-16
            rec = 1.0 / den
            for c in range(C):
                cv = jnp.full((16,), c, I32)
                bc = plsc.load_gather(bias_loc, [cv])
                v = plsc.load_gather(rows_s, [rid, cv]) * rec + bc
                plsc.store_scatter(rows_g, [rid, cv], jnp.maximum(v, 0.0))
        pltpu.sync_copy(rows_g.at[pl.ds(0, GP)], xout_hbm.at[r, pl.ds(rbase, GP)])
        return carry

    lax.fori_loop(0, RPT // GP, _post, 0)


def _stack2(pa, pb, k):
    return jnp.stack([pa[k], pb[k]])


def _stack2v(pa, pb, k):
    return jnp.stack([pa[k], pb[k]])[:, None, :]


def kernel(x_user, x_item, edge_index_u2i, edge_index_i2u, params):
    p = params
    pad = ((0, NP - N), (0, 0))
    xs = jnp.stack([jnp.pad(x_user, pad), jnp.pad(x_item, pad)])

    src = jnp.stack([edge_index_u2i[0], edge_index_i2u[0]])
    dst = jnp.stack([edge_index_u2i[1], edge_index_i2u[1]])
    zeros = jnp.zeros((NP, W), F32)

    c0u, c0i = p['c0_u2i'], p['c0_i2u']
    c1u, c1i = p['c1_u2i'], p['c1_i2u']

    hs0, asrc0, adst0 = _make_proj(D_IN, 25, False)(
        xs, xs, _stack2(c0u, c0i, 'Ws'), _stack2v(c0u, c0i, 'as'),
        _stack2(c0u, c0i, 'Wd'), _stack2v(c0u, c0i, 'ad'))
    x1 = _edge_kernel(src, dst, asrc0.reshape(2, NP), adst0.reshape(2, NP),
                      _stack2(c0u, c0i, 'b'), hs0.reshape(2 * NP, C), zeros)

    hs1, asrc1, adst1 = _make_proj(C, 25, True)(
        x1, x1, _stack2(c1u, c1i, 'Ws'), _stack2v(c1u, c1i, 'as'),
        _stack2(c1u, c1i, 'Wd'), _stack2v(c1u, c1i, 'ad'))
    x2 = _edge_kernel(src, dst, asrc1.reshape(2, NP), adst1.reshape(2, NP),
                      _stack2(c1u, c1i, 'b'), hs1.reshape(2 * NP, C), zeros)

    outs, = _make_final(25)(
        x2,
        jnp.stack([p['lin0_u_W'], p['lin0_i_W']]),
        jnp.stack([p['lin0_u_b'], p['lin0_i_b']])[:, None, :],
        jnp.stack([p['lin1_u_W'], p['lin1_i_W']]),
        jnp.stack([p['lin1_u_b'], p['lin1_i_b']])[:, None, :])
    return (outs[0, :N], outs[1, :N])
